# Initial kernel scaffold; baseline (speedup 1.0000x reference)
import jax
import jax.numpy as jnp
from jax.experimental import pallas as pl


def kernel(positions, encodings):
    raise NotImplementedError("write your pallas kernel here")


# speedup vs baseline: 7.1580x; 7.1580x over previous
import functools

import jax
import jax.numpy as jnp
from jax import lax
from jax.experimental import pallas as pl
from jax.experimental.pallas import tpu as pltpu
from jax.experimental.pallas import tpu_sc as plsc

DIM = 128
NC, NS = 2, 16
NW = NC * NS
CHUNK = 128
B_TOTAL = 4096 * 50
B_PER_W = B_TOTAL // NW
NSTEP = B_PER_W // CHUNK

_mesh = plsc.VectorSubcoreMesh(core_axis_name="c", subcore_axis_name="s")


@functools.partial(
    pl.kernel,
    out_type=jax.ShapeDtypeStruct((B_TOTAL, DIM), jnp.float32),
    mesh=_mesh,
    scratch_types=[
        pltpu.VMEM((NSTEP, CHUNK), jnp.int32),
        pltpu.VMEM((CHUNK, DIM), jnp.float32),
        pltpu.SemaphoreType.DMA,
    ],
)
def _gather(enc_hbm, pos_hbm, out_hbm, idx_v, rows_v, sem):
    wid = lax.axis_index("s") * NC + lax.axis_index("c")
    base = wid * B_PER_W
    pltpu.sync_copy(pos_hbm.at[wid], idx_v)

    @pl.loop(0, NSTEP)
    def _(j):
        pltpu.async_copy(enc_hbm.at[idx_v.at[j]], rows_v, sem).wait()
        pltpu.sync_copy(rows_v, out_hbm.at[pl.ds(base + j * CHUNK, CHUNK)])


def kernel(positions, encodings):
    pos = positions.reshape(NW, NSTEP, CHUNK)
    out = _gather(encodings, pos)
    return out.reshape(4096, 1, 50, DIM)


# speedup vs baseline: 9.4609x; 1.3217x over previous
import functools

import jax
import jax.numpy as jnp
from jax import lax
from jax.experimental import pallas as pl
from jax.experimental.pallas import tpu as pltpu
from jax.experimental.pallas import tpu_sc as plsc

DIM = 128
NC, NS = 2, 16
NW = NC * NS
CHUNK = 128
B_TOTAL = 4096 * 50
B_PER_W = B_TOTAL // NW
NSTEP = B_PER_W // CHUNK

_mesh = plsc.VectorSubcoreMesh(core_axis_name="c", subcore_axis_name="s")


@functools.partial(
    pl.kernel,
    out_type=jax.ShapeDtypeStruct((B_TOTAL, DIM), jnp.float32),
    mesh=_mesh,
    scratch_types=[
        pltpu.VMEM((NSTEP, CHUNK), jnp.int32),
        pltpu.VMEM((2, CHUNK, DIM), jnp.float32),
        pltpu.SemaphoreType.DMA,
        pltpu.SemaphoreType.DMA,
        pltpu.SemaphoreType.DMA,
        pltpu.SemaphoreType.DMA,
    ],
)
def _gather(enc_hbm, pos_hbm, out_hbm, idx_v, rows_v, g0, g1, s0, s1):
    wid = lax.axis_index("s") * NC + lax.axis_index("c")
    base = wid * B_PER_W
    pltpu.sync_copy(pos_hbm.at[wid], idx_v)

    gsem = (g0, g1)
    ssem = (s0, s1)

    def start_gather(j, b):
        pltpu.async_copy(enc_hbm.at[idx_v.at[j]], rows_v.at[b], gsem[b])

    def wait_gather(j, b):
        pltpu.make_async_copy(enc_hbm.at[idx_v.at[j]], rows_v.at[b], gsem[b]).wait()

    def out_slice(j):
        return out_hbm.at[pl.ds(base + j * CHUNK, CHUNK)]

    start_gather(0, 0)
    start_gather(1, 1)

    @pl.loop(0, NSTEP - 2, step=2)
    def _(t):
        for b in range(2):
            j = t + b
            wait_gather(j, b)
            pltpu.async_copy(rows_v.at[b], out_slice(j), ssem[b]).wait()
            start_gather(j + 2, b)

    for b in range(2):
        j = NSTEP - 2 + b
        wait_gather(j, b)
        pltpu.async_copy(rows_v.at[b], out_slice(j), ssem[b]).wait()


def kernel(positions, encodings):
    pos = positions.reshape(NW, NSTEP, CHUNK)
    out = _gather(encodings, pos)
    return out.reshape(4096, 1, 50, DIM)


# speedup vs baseline: 9.5671x; 1.0112x over previous
import functools

import jax
import jax.numpy as jnp
from jax import lax
from jax.experimental import pallas as pl
from jax.experimental.pallas import tpu as pltpu
from jax.experimental.pallas import tpu_sc as plsc

DIM = 128
NC, NS = 2, 16
NW = NC * NS
CHUNK = 128
B_TOTAL = 4096 * 50
B_PER_W = B_TOTAL // NW
NSTEP = B_PER_W // CHUNK

_mesh = plsc.VectorSubcoreMesh(core_axis_name="c", subcore_axis_name="s")


NBUF = 5


@functools.partial(
    pl.kernel,
    out_type=jax.ShapeDtypeStruct((B_TOTAL, DIM), jnp.float32),
    mesh=_mesh,
    scratch_types=[
        pltpu.VMEM((NSTEP, CHUNK), jnp.int32),
        pltpu.VMEM((NBUF, CHUNK, DIM), jnp.float32),
        [pltpu.SemaphoreType.DMA] * NBUF,
        [pltpu.SemaphoreType.DMA] * NBUF,
    ],
)
def _gather(enc_hbm, pos_hbm, out_hbm, idx_v, rows_v, gsem, ssem):
    wid = lax.axis_index("s") * NC + lax.axis_index("c")
    base = wid * B_PER_W
    pltpu.sync_copy(pos_hbm.at[wid], idx_v)

    def sg(j, b):
        pltpu.async_copy(enc_hbm.at[idx_v.at[j]], rows_v.at[b], gsem[b])

    def wg(j, b):
        pltpu.make_async_copy(enc_hbm.at[idx_v.at[j]], rows_v.at[b], gsem[b]).wait()

    def out_slice(j):
        return out_hbm.at[pl.ds(base + j * CHUNK, CHUNK)]

    def ss(j, b):
        pltpu.async_copy(rows_v.at[b], out_slice(j), ssem[b])

    def ws(j, b):
        pltpu.make_async_copy(rows_v.at[b], out_slice(j), ssem[b]).wait()

    sg(0, 0)
    sg(1, 1)

    for j in range(NBUF):
        bb = (j + 2) % NBUF
        if j + 2 >= NBUF:
            ws(j - 3, bb)
        sg(j + 2, bb)
        wg(j, j)
        ss(j, j)

    @pl.loop(NBUF, NSTEP - NBUF, step=NBUF)
    def _(t):
        for b in range(NBUF):
            j = t + b
            bb = (b + 2) % NBUF
            ws(j - 3, bb)
            sg(j + 2, bb)
            wg(j, b)
            ss(j, b)

    for b in range(NBUF):
        j = NSTEP - NBUF + b
        bb = (b + 2) % NBUF
        if j + 2 < NSTEP:
            ws(j - 3, bb)
            sg(j + 2, bb)
        wg(j, b)
        ss(j, b)

    for b in range(NBUF):
        ws(NSTEP - NBUF + b, b)


def kernel(positions, encodings):
    pos = positions.reshape(NW, NSTEP, CHUNK)
    out = _gather(encodings, pos)
    return out.reshape(4096, 1, 50, DIM)


# speedup vs baseline: 14.1281x; 1.4767x over previous
import functools

import jax
import jax.numpy as jnp
from jax import lax
from jax.experimental import pallas as pl
from jax.experimental.pallas import tpu as pltpu
from jax.experimental.pallas import tpu_sc as plsc

DIM = 128
NC, NS = 2, 16
NW = NC * NS
CHUNK = 64
B_TOTAL = 4096 * 50
B_PER_W = B_TOTAL // NW
NSTEP = B_PER_W // CHUNK

_mesh = plsc.VectorSubcoreMesh(core_axis_name="c", subcore_axis_name="s")


NBUF = 5


@functools.partial(
    pl.kernel,
    out_type=jax.ShapeDtypeStruct((B_TOTAL, DIM), jnp.float32),
    mesh=_mesh,
    scratch_types=[
        pltpu.VMEM((NSTEP, CHUNK), jnp.int32),
        pltpu.VMEM((NBUF, CHUNK, DIM), jnp.float32),
        pltpu.VMEM_SHARED((8192, DIM), jnp.float32),
        [pltpu.SemaphoreType.DMA] * NBUF,
        [pltpu.SemaphoreType.DMA] * NBUF,
    ],
)
def _gather(enc_hbm, pos_hbm, out_hbm, idx_v, rows_v, enc_sh, gsem, ssem):
    wid = lax.axis_index("s") * NC + lax.axis_index("c")
    base = wid * B_PER_W

    sid = lax.axis_index("s")
    stage = 8192 // NS
    pltpu.sync_copy(enc_hbm.at[pl.ds(sid * stage, stage)],
                    enc_sh.at[pl.ds(sid * stage, stage)])
    pltpu.sync_copy(pos_hbm.at[wid], idx_v)
    plsc.subcore_barrier()

    def sg(j, b):
        pltpu.async_copy(enc_sh.at[idx_v.at[j]], rows_v.at[b], gsem[b])

    def wg(j, b):
        pltpu.make_async_copy(enc_sh.at[idx_v.at[j]], rows_v.at[b], gsem[b]).wait()

    def out_slice(j):
        return out_hbm.at[pl.ds(base + j * CHUNK, CHUNK)]

    def ss(j, b):
        pltpu.async_copy(rows_v.at[b], out_slice(j), ssem[b])

    def ws(j, b):
        pltpu.make_async_copy(rows_v.at[b], out_slice(j), ssem[b]).wait()

    sg(0, 0)
    sg(1, 1)

    for j in range(NBUF):
        bb = (j + 2) % NBUF
        if j + 2 >= NBUF:
            ws(j - 3, bb)
        sg(j + 2, bb)
        wg(j, j)
        ss(j, j)

    @pl.loop(NBUF, NSTEP - NBUF, step=NBUF)
    def _(t):
        for b in range(NBUF):
            j = t + b
            bb = (b + 2) % NBUF
            ws(j - 3, bb)
            sg(j + 2, bb)
            wg(j, b)
            ss(j, b)

    for b in range(NBUF):
        j = NSTEP - NBUF + b
        bb = (b + 2) % NBUF
        if j + 2 < NSTEP:
            ws(j - 3, bb)
            sg(j + 2, bb)
        wg(j, b)
        ss(j, b)

    for b in range(NBUF):
        ws(NSTEP - NBUF + b, b)


def kernel(positions, encodings):
    pos = positions.reshape(NW, NSTEP, CHUNK)
    out = _gather(encodings, pos)
    return out.reshape(4096, 1, 50, DIM)


# speedup vs baseline: 14.4629x; 1.0237x over previous
import functools

import jax
import jax.numpy as jnp
from jax import lax
from jax.experimental import pallas as pl
from jax.experimental.pallas import tpu as pltpu
from jax.experimental.pallas import tpu_sc as plsc

DIM = 128
NC, NS = 2, 16
NW = NC * NS
CHUNK = 80
B_TOTAL = 4096 * 50
B_PER_W = B_TOTAL // NW
NSTEP = B_PER_W // CHUNK

_mesh = plsc.VectorSubcoreMesh(core_axis_name="c", subcore_axis_name="s")


NBUF = 4


@functools.partial(
    pl.kernel,
    out_type=jax.ShapeDtypeStruct((B_TOTAL, DIM), jnp.float32),
    mesh=_mesh,
    scratch_types=[
        pltpu.VMEM((NSTEP, CHUNK), jnp.int32),
        pltpu.VMEM((NBUF, CHUNK, DIM), jnp.float32),
        pltpu.VMEM_SHARED((8192, DIM), jnp.float32),
        [pltpu.SemaphoreType.DMA] * NBUF,
        [pltpu.SemaphoreType.DMA] * NBUF,
    ],
)
def _gather(enc_hbm, pos_hbm, out_hbm, idx_v, rows_v, enc_sh, gsem, ssem):
    wid = lax.axis_index("s") * NC + lax.axis_index("c")
    base = wid * B_PER_W

    sid = lax.axis_index("s")
    stage = 8192 // NS
    pltpu.sync_copy(enc_hbm.at[pl.ds(sid * stage, stage)],
                    enc_sh.at[pl.ds(sid * stage, stage)])
    pltpu.sync_copy(pos_hbm.at[wid], idx_v)
    plsc.subcore_barrier()

    def sg(j, b):
        pltpu.async_copy(enc_sh.at[idx_v.at[j]], rows_v.at[b], gsem[b])

    def wg(j, b):
        pltpu.make_async_copy(enc_sh.at[idx_v.at[j]], rows_v.at[b], gsem[b]).wait()

    def out_slice(j):
        return out_hbm.at[pl.ds(base + j * CHUNK, CHUNK)]

    def ss(j, b):
        pltpu.async_copy(rows_v.at[b], out_slice(j), ssem[b])

    def ws(j, b):
        pltpu.make_async_copy(rows_v.at[b], out_slice(j), ssem[b]).wait()

    sg(0, 0)
    sg(1, 1)

    for j in range(NBUF):
        bb = (j + 2) % NBUF
        if j + 2 >= NBUF:
            ws(j - 3, bb)
        sg(j + 2, bb)
        wg(j, j)
        ss(j, j)

    @pl.loop(NBUF, NSTEP - NBUF, step=NBUF)
    def _(t):
        for b in range(NBUF):
            j = t + b
            bb = (b + 2) % NBUF
            ws(j - 3, bb)
            sg(j + 2, bb)
            wg(j, b)
            ss(j, b)

    for b in range(NBUF):
        j = NSTEP - NBUF + b
        bb = (b + 2) % NBUF
        if j + 2 < NSTEP:
            ws(j - 3, bb)
            sg(j + 2, bb)
        wg(j, b)
        ss(j, b)

    for b in range(NBUF):
        ws(NSTEP - NBUF + b, b)


def kernel(positions, encodings):
    pos = positions.reshape(NW, NSTEP, CHUNK)
    out = _gather(encodings, pos)
    return out.reshape(4096, 1, 50, DIM)
